# baseline (speedup 1.0000x reference)
import functools

import jax
import jax.numpy as jnp
from jax import lax
from jax.experimental import pallas as pl
from jax.experimental.pallas import tpu as pltpu
from jax.experimental.pallas import tpu_sc as plsc

B = 64
V = 100000
L = 16
NB = 512
RANGE = 16.0
INV_DELTA = NB / RANGE
TOP_P = 0.95
NW = 32
ROWS_PER_W = B // NW
VECS = V // L
CHUNK = 10000
NCH = V // CHUNK
CVECS = CHUNK // L


def _lane_reduce(vec, op):
    idx = lax.iota(jnp.int32, L)
    for sh in (8, 4, 2, 1):
        perm = jnp.bitwise_xor(idx, sh)
        vec = op(vec, vec.at[perm].get(mode="promise_in_bounds"))
    return vec


def _sc_kernel(logits_hbm, he_hbm, ht_hbm, idx_hbm, row_v, he_v, ht_v,
               si_v, sem0, sem1):
    wid = lax.axis_index("s") * 2 + lax.axis_index("c")
    iota = lax.iota(jnp.int32, L)
    iota_nb = iota * NB
    zeros = jnp.zeros((L,), jnp.float32)

    sems = (sem0, sem1)
    for rr in range(ROWS_PER_W):
        r = wid * ROWS_PER_W + rr

        rbase = r * V
        handles = [None, None]
        for c in range(min(2, NCH)):
            handles[c % 2] = pltpu.async_copy(
                logits_hbm.at[pl.ds(rbase + c * CHUNK, CHUNK)],
                row_v.at[pl.ds(c * CHUNK, CHUNK)], sems[c % 2])

        @plsc.parallel_loop(0, NB * L // L, unroll=8)
        def _(j):
            he_v[pl.ds(j * L, L)] = zeros
            ht_v[pl.ds(j * L, L)] = zeros

        acc_v0 = jnp.full((L,), -jnp.inf, jnp.float32)
        acc_i0 = jnp.zeros((L,), jnp.int32)
        acc = (acc_v0, acc_i0)
        for c in range(NCH):
            handles[c % 2].wait()
            if c + 2 < NCH:
                handles[c % 2] = pltpu.async_copy(
                    logits_hbm.at[pl.ds(rbase + (c + 2) * CHUNK, CHUNK)],
                    row_v.at[pl.ds((c + 2) * CHUNK, CHUNK)], sems[c % 2])
            base0 = c * CHUNK

            @plsc.parallel_loop(0, CVECS, unroll=8, carry=acc)
            def max_loop(i, carry):
                acc_v, acc_i = carry
                x = row_v[pl.ds(base0 + i * L, L)]
                gi = iota + (base0 + i * L)
                upd = x > acc_v
                acc_i = jnp.where(upd, gi, acc_i)
                acc_v = jnp.where(upd, x, acc_v)
                return acc_v, acc_i

            acc = max_loop

        acc_v, acc_i = acc
        mv = _lane_reduce(acc_v, jnp.maximum)
        cand = jnp.where(acc_v == mv, acc_i, jnp.int32(2**31 - 1))
        gvec = _lane_reduce(cand, jnp.minimum)

        @plsc.parallel_loop(0, VECS, unroll=8)
        def _(i):
            x = row_v[pl.ds(i * L, L)]
            y = x - mv
            e = jnp.exp(y)
            f = jnp.minimum(y * (-INV_DELTA), float(NB - 1))
            flat = f.astype(jnp.int32) + iota_nb
            plsc.addupdate_scatter(he_v, [flat], e)
            plsc.addupdate_scatter(ht_v, [flat], e * y)

        pltpu.sync_copy(he_v, he_hbm.at[r])
        pltpu.sync_copy(ht_v, ht_hbm.at[r])
        si_v[...] = gvec
        pltpu.sync_copy(si_v, idx_hbm.at[r])


_sc_call = functools.partial(
    pl.kernel,
    out_type=[
        jax.ShapeDtypeStruct((B, L * NB), jnp.float32),
        jax.ShapeDtypeStruct((B, L * NB), jnp.float32),
        jax.ShapeDtypeStruct((B, L), jnp.int32),
    ],
    mesh=plsc.VectorSubcoreMesh(core_axis_name="c", subcore_axis_name="s"),
    compiler_params=pltpu.CompilerParams(needs_layout_passes=False),
    scratch_types=[
        pltpu.VMEM((V,), jnp.float32),
        pltpu.VMEM((L * NB,), jnp.float32),
        pltpu.VMEM((L * NB,), jnp.float32),
        pltpu.VMEM((L,), jnp.int32),
        pltpu.SemaphoreType.DMA,
        pltpu.SemaphoreType.DMA,
    ],
)(_sc_kernel)


def _finisher(he_ref, ht_ref, conf_ref):
    heb = he_ref[:, 0:NB]
    htb = ht_ref[:, 0:NB]
    for l in range(1, L):
        heb = heb + he_ref[:, l * NB:(l + 1) * NB]
        htb = htb + ht_ref[:, l * NB:(l + 1) * NB]
    z = jnp.sum(heb, axis=-1, keepdims=True)
    rix = lax.broadcasted_iota(jnp.int32, (NB, NB), 0)
    cix = lax.broadcasted_iota(jnp.int32, (NB, NB), 1)
    tri = (rix < cix).astype(jnp.float32)
    cumex = jnp.dot(heb, tri, preferred_element_type=jnp.float32)
    kept = cumex <= TOP_P * z
    s = jnp.sum(jnp.where(kept, heb, 0.0), axis=-1, keepdims=True)
    t = jnp.sum(jnp.where(kept, htb, 0.0), axis=-1, keepdims=True)
    conf_ref[...] = t / s - jnp.log(s)


def kernel(logits):
    assert logits.shape == (B, V) and logits.dtype == jnp.float32
    he, ht, idx = _sc_call(logits.reshape(-1))
    conf2 = pl.pallas_call(
        _finisher,
        out_shape=jax.ShapeDtypeStruct((B, 1), jnp.float32),
    )(he, ht)
    return conf2.reshape(B), idx[:, 0]


# speedup vs baseline: 1.5498x; 1.5498x over previous
import functools

import jax
import jax.numpy as jnp
from jax import lax
from jax.experimental import pallas as pl
from jax.experimental.pallas import tpu as pltpu
from jax.experimental.pallas import tpu_sc as plsc

B = 64
V = 100000
L = 16
NB = 1024
RANGE = 16.0
INV_DELTA = NB / RANGE
DELTA = RANGE / NB
TOP_P = 0.95
NW = 32
ROWS_PER_W = B // NW
VECS = V // L


def _lane_reduce(vec, op):
    idx = lax.iota(jnp.int32, L)
    for sh in (8, 4, 2, 1):
        perm = jnp.bitwise_xor(idx, sh)
        vec = op(vec, vec.at[perm].get(mode="promise_in_bounds"))
    return vec


def _sc_kernel(logits_hbm, hc_hbm, idx_hbm, row_v, hc_v, si_v):
    wid = lax.axis_index("s") * 2 + lax.axis_index("c")
    iota = lax.iota(jnp.int32, L)
    iota_nb = iota * NB
    zeros = jnp.zeros((L,), jnp.float32)
    ones = jnp.ones((L,), jnp.float32)

    for rr in range(ROWS_PER_W):
        r = wid * ROWS_PER_W + rr
        pltpu.sync_copy(logits_hbm.at[r], row_v)

        @plsc.parallel_loop(0, NB, unroll=8)
        def _(j):
            hc_v[pl.ds(j * L, L)] = zeros

        acc_v0 = jnp.full((L,), -jnp.inf, jnp.float32)
        acc_i0 = jnp.zeros((L,), jnp.int32)

        @plsc.parallel_loop(0, VECS, unroll=8, carry=(acc_v0, acc_i0))
        def max_loop(i, carry):
            acc_v, acc_i = carry
            x = row_v[pl.ds(i * L, L)]
            gi = iota + i * L
            upd = x > acc_v
            acc_i = jnp.where(upd, gi, acc_i)
            acc_v = jnp.where(upd, x, acc_v)
            return acc_v, acc_i

        acc_v, acc_i = max_loop
        mv = _lane_reduce(acc_v, jnp.maximum)
        cand = jnp.where(acc_v == mv, acc_i, jnp.int32(2**31 - 1))
        gvec = _lane_reduce(cand, jnp.minimum)

        @plsc.parallel_loop(0, VECS, unroll=8)
        def _(i):
            x = row_v[pl.ds(i * L, L)]
            f = jnp.minimum((mv - x) * INV_DELTA, float(NB - 1))
            flat = f.astype(jnp.int32) + iota_nb
            plsc.addupdate_scatter(hc_v, [flat], ones)

        pltpu.sync_copy(hc_v, hc_hbm.at[r])
        si_v[...] = gvec
        pltpu.sync_copy(si_v, idx_hbm.at[r])


_sc_call = functools.partial(
    pl.kernel,
    out_type=[
        jax.ShapeDtypeStruct((B, L * NB), jnp.float32),
        jax.ShapeDtypeStruct((B, L), jnp.int32),
    ],
    mesh=plsc.VectorSubcoreMesh(core_axis_name="c", subcore_axis_name="s"),
    compiler_params=pltpu.CompilerParams(needs_layout_passes=False),
    scratch_types=[
        pltpu.VMEM((V,), jnp.float32),
        pltpu.VMEM((L * NB,), jnp.float32),
        pltpu.VMEM((L,), jnp.int32),
    ],
)(_sc_kernel)


def _finisher(hc_ref, conf_ref):
    cnt = hc_ref[:, 0:NB]
    for l in range(1, L):
        cnt = cnt + hc_ref[:, l * NB:(l + 1) * NB]
    centers = (lax.broadcasted_iota(jnp.int32, (1, NB), 1).astype(jnp.float32)
               + 0.5) * (-DELTA)
    heb = cnt * jnp.exp(centers)
    htb = heb * centers
    z = jnp.sum(heb, axis=-1, keepdims=True)
    rix = lax.broadcasted_iota(jnp.int32, (NB, NB), 0)
    cix = lax.broadcasted_iota(jnp.int32, (NB, NB), 1)
    tri = (rix < cix).astype(jnp.float32)
    cumex = jnp.dot(heb, tri, preferred_element_type=jnp.float32)
    kept = cumex <= TOP_P * z
    s = jnp.sum(jnp.where(kept, heb, 0.0), axis=-1, keepdims=True)
    t = jnp.sum(jnp.where(kept, htb, 0.0), axis=-1, keepdims=True)
    conf_ref[...] = t / s - jnp.log(s)


def kernel(logits):
    assert logits.shape == (B, V) and logits.dtype == jnp.float32
    hc, idx = _sc_call(logits)
    conf2 = pl.pallas_call(
        _finisher,
        out_shape=jax.ShapeDtypeStruct((B, 1), jnp.float32),
    )(hc)
    return conf2.reshape(B), idx[:, 0]


# speedup vs baseline: 1.5898x; 1.0258x over previous
import functools

import jax
import jax.numpy as jnp
from jax import lax
from jax.experimental import pallas as pl
from jax.experimental.pallas import tpu as pltpu
from jax.experimental.pallas import tpu_sc as plsc

B = 64
V = 100000
L = 16
NB = 1024
RANGE = 16.0
INV_DELTA = NB / RANGE
DELTA = RANGE / NB
TOP_P = 0.95
NW = 32
ROWS_PER_W = B // NW
VECS = V // L
HALF = V // 2
HVECS = HALF // L


def _lane_reduce(vec, op):
    idx = lax.iota(jnp.int32, L)
    for sh in (8, 4, 2, 1):
        perm = jnp.bitwise_xor(idx, sh)
        vec = op(vec, vec.at[perm].get(mode="promise_in_bounds"))
    return vec


def _sc_kernel(logits_hbm, conf_hbm, idx_hbm, row_v, hc_v, me_v, si_v, cf_v):
    wid = lax.axis_index("s") * 2 + lax.axis_index("c")
    iota = lax.iota(jnp.int32, L)
    iota_nb = iota * NB
    fiota = iota.astype(jnp.float32)
    zeros = jnp.zeros((L,), jnp.float32)
    ones = jnp.ones((L,), jnp.float32)

    for rr in range(ROWS_PER_W):
        r = wid * ROWS_PER_W + rr
        pltpu.sync_copy(logits_hbm.at[r], row_v)

        @plsc.parallel_loop(0, NB, unroll=8)
        def _(j):
            hc_v[pl.ds(j * L, L)] = zeros

        acc_v0 = jnp.full((L,), -jnp.inf, jnp.float32)
        acc_i0 = jnp.zeros((L,), jnp.int32)

        @plsc.parallel_loop(0, VECS, unroll=8, carry=(acc_v0, acc_i0))
        def max_loop(i, carry):
            acc_v, acc_i = carry
            x = row_v[pl.ds(i * L, L)]
            gi = iota + i * L
            upd = x > acc_v
            acc_i = jnp.where(upd, gi, acc_i)
            acc_v = jnp.where(upd, x, acc_v)
            return acc_v, acc_i

        acc_v, acc_i = max_loop
        mv = _lane_reduce(acc_v, jnp.maximum)
        cand = jnp.where(acc_v == mv, acc_i, jnp.int32(2**31 - 1))
        gvec = _lane_reduce(cand, jnp.minimum)

        @plsc.parallel_loop(0, VECS, unroll=8)
        def _(i):
            x = row_v[pl.ds(i * L, L)]
            f = jnp.minimum((mv - x) * INV_DELTA, float(NB - 1))
            flat = f.astype(jnp.int32) + iota_nb
            plsc.addupdate_scatter(hc_v, [flat], ones)

        cvec0 = (fiota + 0.5) * (-DELTA)

        @plsc.parallel_loop(0, NB // L, unroll=2, carry=(zeros, cvec0))
        def mass_loop(j, carry):
            zacc, cvec = carry
            s = hc_v[pl.ds(j * L, L)]
            for l in range(1, L):
                s = s + hc_v[pl.ds(l * NB + j * L, L)]
            e = s * jnp.exp(cvec)
            me_v[pl.ds(j * L, L)] = e
            return zacc + e, cvec - (DELTA * L)

        z = _lane_reduce(mass_loop[0], jnp.add)
        tau = TOP_P * z

        lane15 = jnp.full((L,), L - 1, jnp.int32)

        @plsc.parallel_loop(0, NB // L, carry=(zeros, zeros, zeros, cvec0))
        def scan_loop(j, carry):
            cum, s_acc, t_acc, cvec = carry
            e = me_v[pl.ds(j * L, L)]
            incl = plsc.cumsum(e)
            pre = (incl - e) + cum
            ek = jnp.where(pre <= tau, e, 0.0)
            tot = incl.at[lane15].get(mode="promise_in_bounds")
            return (cum + tot, s_acc + ek, t_acc + ek * cvec,
                    cvec - (DELTA * L))

        _, s_part, t_part, _ = scan_loop
        s_tot = _lane_reduce(s_part, jnp.add)
        t_tot = _lane_reduce(t_part, jnp.add)

        bits = plsc.bitcast(s_tot, jnp.uint32)
        ex = plsc.bitcast(bits >> jnp.uint32(23), jnp.int32) - 127
        man = plsc.bitcast((bits & jnp.uint32(0x7FFFFF))
                           | jnp.uint32(0x3F800000), jnp.float32)
        y = ex.astype(jnp.float32) * jnp.float32(0.6931472) \
            + (man - 1.0) * jnp.float32(0.6931472)
        for _ in range(3):
            y = y + s_tot * jnp.exp(-y) - 1.0

        cf_v[...] = t_tot / s_tot - y
        pltpu.sync_copy(cf_v, conf_hbm.at[r])
        si_v[...] = gvec
        pltpu.sync_copy(si_v, idx_hbm.at[r])


_sc_call = functools.partial(
    pl.kernel,
    out_type=[
        jax.ShapeDtypeStruct((B, L), jnp.float32),
        jax.ShapeDtypeStruct((B, L), jnp.int32),
    ],
    mesh=plsc.VectorSubcoreMesh(core_axis_name="c", subcore_axis_name="s"),
    compiler_params=pltpu.CompilerParams(needs_layout_passes=False),
    scratch_types=[
        pltpu.VMEM((V,), jnp.float32),
        pltpu.VMEM((L * NB,), jnp.float32),
        pltpu.VMEM((NB,), jnp.float32),
        pltpu.VMEM((L,), jnp.int32),
        pltpu.VMEM((L,), jnp.float32),
    ],
)(_sc_kernel)


def kernel(logits):
    assert logits.shape == (B, V) and logits.dtype == jnp.float32
    conf2, idx = _sc_call(logits)
    return conf2[:, 0], idx[:, 0]
